# baseline (speedup 1.0000x reference)
import functools
import math

import jax
import jax.numpy as jnp
from jax.experimental import pallas as pl
from jax.experimental.pallas import tpu as pltpu
from jax.experimental.pallas import tpu_sc as plsc

D_MODEL = 2048
SKILL_DIM = 128
N_SKILLS = 100000
TOPK = 8
TILE = 2048
N_PAD = ((N_SKILLS + TILE - 1) // TILE) * TILE
NUM_TILES = N_PAD // TILE
B = 1024

_NEG = float("-inf")
_BIGI = 2**30


def _dot_t(a, b):
    return jax.lax.dot_general(
        a.astype(jnp.bfloat16), b.astype(jnp.bfloat16),
        (((1,), (1,)), ((), ())), preferred_element_type=jnp.float32)


def _q_proj_kernel(h_ref, wq_ref, bq_ref, q_ref):
    q_ref[...] = _dot_t(h_ref[...], wq_ref[...]) + bq_ref[...]


def _extract8(vals, idxs=None):
    pos = jax.lax.broadcasted_iota(jnp.int32, vals.shape, 1)
    out_v, out_i = [], []
    v = vals
    for t in range(TOPK):
        m = jnp.max(v, axis=1, keepdims=True)
        p = jnp.min(jnp.where(v == m, pos, _BIGI), axis=1, keepdims=True)
        out_v.append(m)
        if idxs is None:
            out_i.append(p)
        else:
            out_i.append(jnp.min(jnp.where(pos == p, idxs, _BIGI),
                                 axis=1, keepdims=True))
        if t != TOPK - 1:
            v = jnp.where(pos == p, _NEG, v)
    return jnp.concatenate(out_v, axis=1), jnp.concatenate(out_i, axis=1)


B_BLK = 512


def _topk_kernel(q_ref, emb_ref, wk_ref, bk_ref, vals_ref, idx_ref,
                 rv_ref, ri_ref):
    j = pl.program_id(1)

    @pl.when(j == 0)
    def _():
        rv_ref[...] = jnp.full((B_BLK, TOPK), _NEG, jnp.float32)
        ri_ref[...] = jnp.zeros((B_BLK, TOPK), jnp.int32)

    keys = _dot_t(emb_ref[...], wk_ref[...]) + bk_ref[...]
    s = _dot_t(q_ref[...], keys)

    local = jax.lax.broadcasted_iota(jnp.int32, s.shape, 1)
    limit = N_SKILLS - j * TILE
    s = jnp.where(local < limit, s, _NEG)

    t_vals, t_pos = _extract8(s)
    t_idx = t_pos + j * TILE

    cv = jnp.concatenate([rv_ref[...], t_vals], axis=1)
    ci = jnp.concatenate([ri_ref[...], t_idx], axis=1)
    mv, mi = _extract8(cv, ci)
    rv_ref[...] = mv
    ri_ref[...] = mi

    @pl.when(j == NUM_TILES - 1)
    def _():
        vals_ref[...] = mv * (1.0 / math.sqrt(SKILL_DIM))
        idx_ref[...] = mi


def _combine_kernel(tv_ref, g_ref, out_ref):
    tv = tv_ref[...]
    m = jnp.max(tv, axis=1, keepdims=True)
    e = jnp.exp(tv - m)
    w = e / jnp.sum(e, axis=1, keepdims=True)
    acc = jnp.zeros((B, SKILL_DIM), jnp.float32)
    for k in range(TOPK):
        acc = acc + w[:, k:k + 1] * g_ref[:, k, :]
    out_ref[...] = acc


def _sc_gather(table, indices):
    n_idx = indices.shape[1]
    window = 128
    mesh = plsc.VectorSubcoreMesh(core_axis_name="core",
                                  subcore_axis_name="subcore")

    @pl.kernel(out_type=jax.ShapeDtypeStruct((n_idx, SKILL_DIM), table.dtype),
               mesh=mesh)
    def gather_kernel(x_hbm, i_hbm, o_hbm):
        def body(i_vmem, o_vmem):
            pltpu.sync_copy(x_hbm.at[i_vmem.at[0]], o_vmem)

        pltpu.emit_pipeline(
            body,
            grid=(n_idx // window,),
            in_specs=[pl.BlockSpec((1, window), index_map=lambda i: (0, i))],
            out_specs=[pl.BlockSpec((window, SKILL_DIM),
                                    index_map=lambda i: (i, 0))],
            core_axis_name=("core", "subcore"),
            dimension_semantics=(pltpu.PARALLEL,),
        )(i_hbm, o_hbm)

    return gather_kernel(table, indices)


@jax.jit
def kernel(h, skill_embeds, Wq, bq, Wk, bk):
    h_last = h[:, -1, :]
    bq2 = bq.reshape(1, SKILL_DIM)
    bk2 = bk.reshape(1, SKILL_DIM)

    q = pl.pallas_call(
        _q_proj_kernel,
        out_shape=jax.ShapeDtypeStruct((B, SKILL_DIM), jnp.float32),
    )(h_last, Wq, bq2)

    emb_pad = jnp.pad(skill_embeds, ((0, N_PAD - N_SKILLS), (0, 0)))

    top_vals, top_idx = pl.pallas_call(
        _topk_kernel,
        grid=(B // B_BLK, NUM_TILES),
        in_specs=[
            pl.BlockSpec((B_BLK, SKILL_DIM), lambda b, j: (b, 0)),
            pl.BlockSpec((TILE, SKILL_DIM), lambda b, j: (j, 0)),
            pl.BlockSpec((SKILL_DIM, SKILL_DIM), lambda b, j: (0, 0)),
            pl.BlockSpec((1, SKILL_DIM), lambda b, j: (0, 0)),
        ],
        out_specs=[
            pl.BlockSpec((B_BLK, TOPK), lambda b, j: (b, 0)),
            pl.BlockSpec((B_BLK, TOPK), lambda b, j: (b, 0)),
        ],
        out_shape=[
            jax.ShapeDtypeStruct((B, TOPK), jnp.float32),
            jax.ShapeDtypeStruct((B, TOPK), jnp.int32),
        ],
        scratch_shapes=[
            pltpu.VMEM((B_BLK, TOPK), jnp.float32),
            pltpu.VMEM((B_BLK, TOPK), jnp.int32),
        ],
        compiler_params=pltpu.CompilerParams(
            dimension_semantics=("parallel", "arbitrary")),
    )(q, emb_pad, Wk, bk2)

    gathered = _sc_gather(skill_embeds, top_idx.reshape(1, B * TOPK))

    retrieved = pl.pallas_call(
        _combine_kernel,
        out_shape=jax.ShapeDtypeStruct((B, SKILL_DIM), jnp.float32),
    )(top_vals, gathered.reshape(B, TOPK, SKILL_DIM))

    return (retrieved, top_idx)


# speedup vs baseline: 1.4987x; 1.4987x over previous
import functools
import math

import jax
import jax.numpy as jnp
from jax.experimental import pallas as pl
from jax.experimental.pallas import tpu as pltpu
from jax.experimental.pallas import tpu_sc as plsc

D_MODEL = 2048
SKILL_DIM = 128
N_SKILLS = 100000
TOPK = 8
TILE = 2048
N_PAD = ((N_SKILLS + TILE - 1) // TILE) * TILE
NUM_TILES = N_PAD // TILE
B = 1024

_NEG = float("-inf")
_BIGI = 2**30


def _dot_t(a, b):
    return jax.lax.dot_general(
        a.astype(jnp.bfloat16), b.astype(jnp.bfloat16),
        (((1,), (1,)), ((), ())), preferred_element_type=jnp.float32)


def _q_proj_kernel(h_ref, wq_ref, bq_ref, q_ref):
    q_ref[...] = _dot_t(h_ref[...], wq_ref[...]) + bq_ref[...]


def _extract8(vals, idxs=None):
    pos = jax.lax.broadcasted_iota(jnp.int32, vals.shape, 1)
    out_v, out_i = [], []
    v = vals
    for t in range(TOPK):
        m = jnp.max(v, axis=1, keepdims=True)
        p = jnp.min(jnp.where(v == m, pos, _BIGI), axis=1, keepdims=True)
        out_v.append(m)
        if idxs is None:
            out_i.append(p)
        else:
            out_i.append(jnp.min(jnp.where(pos == p, idxs, _BIGI),
                                 axis=1, keepdims=True))
        if t != TOPK - 1:
            v = jnp.where(pos == p, _NEG, v)
    return jnp.concatenate(out_v, axis=1), jnp.concatenate(out_i, axis=1)


NGRP = TILE // SKILL_DIM


def _merge8(rv, ri, t_vals, t_idx):
    cv = jnp.concatenate([rv, t_vals], axis=1)
    ci = jnp.concatenate([ri, t_idx], axis=1)
    return _extract8(cv, ci)


def _topk_kernel(q_ref, emb_ref, wk_ref, bk_ref, vals_ref, idx_ref,
                 rv_ref, ri_ref):
    j = pl.program_id(0)

    @pl.when(j == 0)
    def _():
        rv_ref[...] = jnp.full((B, TOPK), _NEG, jnp.float32)
        ri_ref[...] = jnp.zeros((B, TOPK), jnp.int32)

    keys = _dot_t(emb_ref[...], wk_ref[...]) + bk_ref[...]
    s = _dot_t(q_ref[...], keys)

    local = jax.lax.broadcasted_iota(jnp.int32, s.shape, 1)
    limit = N_SKILLS - j * TILE
    s = jnp.where(local < limit, s, _NEG)

    lane = jax.lax.broadcasted_iota(jnp.int32, (B, SKILL_DIM), 1)
    m1 = s[:, 0:SKILL_DIM]
    g1 = jnp.zeros((B, SKILL_DIM), jnp.int32)
    m2 = jnp.full((B, SKILL_DIM), _NEG, jnp.float32)
    g2 = jnp.zeros((B, SKILL_DIM), jnp.int32)
    for g in range(1, NGRP):
        x = s[:, g * SKILL_DIM:(g + 1) * SKILL_DIM]
        c1 = x > m1
        d = jnp.where(c1, m1, x)
        gd = jnp.where(c1, g1, g)
        m1 = jnp.where(c1, x, m1)
        g1 = jnp.where(c1, g, g1)
        c2 = d > m2
        m2 = jnp.where(c2, d, m2)
        g2 = jnp.where(c2, gd, g2)

    col1 = g1 * SKILL_DIM + lane
    col2 = g2 * SKILL_DIM + lane
    cvals = m1
    ccols = col1
    used = jnp.zeros((B, SKILL_DIM), jnp.int32)
    tv, ti = [], []
    for _ in range(TOPK):
        m = jnp.max(cvals, axis=1, keepdims=True)
        p = jnp.min(jnp.where(cvals == m, ccols, _BIGI), axis=1, keepdims=True)
        tv.append(m)
        ti.append(p)
        sel = (ccols == p) & (cvals == m)
        fresh = used == 0
        cvals = jnp.where(sel, jnp.where(fresh, m2, _NEG), cvals)
        ccols = jnp.where(sel, jnp.where(fresh, col2, ccols), ccols)
        used = used + sel.astype(jnp.int32)
    t_vals = jnp.concatenate(tv, axis=1)
    t_idx = jnp.concatenate(ti, axis=1) + j * TILE

    rv_old = rv_ref[...]
    ri_old = ri_ref[...]
    mv, mi = _merge8(rv_old, ri_old, t_vals, t_idx)

    tau = mv[:, TOPK - 1:TOPK]
    cnt = jnp.zeros((B, SKILL_DIM), jnp.int32)
    for g in range(NGRP):
        x = s[:, g * SKILL_DIM:(g + 1) * SKILL_DIM]
        cnt = cnt + (x >= tau).astype(jnp.int32)
    bad = jnp.max(cnt) >= 3

    @pl.when(jnp.logical_not(bad))
    def _():
        rv_ref[...] = mv
        ri_ref[...] = mi

    @pl.when(bad)
    def _():
        f_vals, f_pos = _extract8(s)
        fv, fi = _merge8(rv_old, ri_old, f_vals, f_pos + j * TILE)
        rv_ref[...] = fv
        ri_ref[...] = fi

    @pl.when(j == NUM_TILES - 1)
    def _():
        vals_ref[...] = rv_ref[...] * (1.0 / math.sqrt(SKILL_DIM))
        idx_ref[...] = ri_ref[...]


def _combine_kernel(tv_ref, g_ref, out_ref):
    tv = tv_ref[...]
    m = jnp.max(tv, axis=1, keepdims=True)
    e = jnp.exp(tv - m)
    w = e / jnp.sum(e, axis=1, keepdims=True)
    acc = jnp.zeros((B, SKILL_DIM), jnp.float32)
    for k in range(TOPK):
        acc = acc + w[:, k:k + 1] * g_ref[:, k, :]
    out_ref[...] = acc


def _sc_gather(table, indices):
    n_idx = indices.shape[1]
    window = 128
    mesh = plsc.VectorSubcoreMesh(core_axis_name="core",
                                  subcore_axis_name="subcore")

    @pl.kernel(out_type=jax.ShapeDtypeStruct((n_idx, SKILL_DIM), table.dtype),
               mesh=mesh)
    def gather_kernel(x_hbm, i_hbm, o_hbm):
        def body(i_vmem, o_vmem):
            pltpu.sync_copy(x_hbm.at[i_vmem.at[0]], o_vmem)

        pltpu.emit_pipeline(
            body,
            grid=(n_idx // window,),
            in_specs=[pl.BlockSpec((1, window), index_map=lambda i: (0, i))],
            out_specs=[pl.BlockSpec((window, SKILL_DIM),
                                    index_map=lambda i: (i, 0))],
            core_axis_name=("core", "subcore"),
            dimension_semantics=(pltpu.PARALLEL,),
        )(i_hbm, o_hbm)

    return gather_kernel(table, indices)


@jax.jit
def kernel(h, skill_embeds, Wq, bq, Wk, bk):
    h_last = h[:, -1, :]
    bq2 = bq.reshape(1, SKILL_DIM)
    bk2 = bk.reshape(1, SKILL_DIM)

    q = pl.pallas_call(
        _q_proj_kernel,
        out_shape=jax.ShapeDtypeStruct((B, SKILL_DIM), jnp.float32),
    )(h_last, Wq, bq2)

    emb_pad = jnp.pad(skill_embeds, ((0, N_PAD - N_SKILLS), (0, 0)))

    top_vals, top_idx = pl.pallas_call(
        _topk_kernel,
        grid=(NUM_TILES,),
        in_specs=[
            pl.BlockSpec((B, SKILL_DIM), lambda j: (0, 0)),
            pl.BlockSpec((TILE, SKILL_DIM), lambda j: (j, 0)),
            pl.BlockSpec((SKILL_DIM, SKILL_DIM), lambda j: (0, 0)),
            pl.BlockSpec((1, SKILL_DIM), lambda j: (0, 0)),
        ],
        out_specs=[
            pl.BlockSpec((B, TOPK), lambda j: (0, 0)),
            pl.BlockSpec((B, TOPK), lambda j: (0, 0)),
        ],
        out_shape=[
            jax.ShapeDtypeStruct((B, TOPK), jnp.float32),
            jax.ShapeDtypeStruct((B, TOPK), jnp.int32),
        ],
        scratch_shapes=[
            pltpu.VMEM((B, TOPK), jnp.float32),
            pltpu.VMEM((B, TOPK), jnp.int32),
        ],
    )(q, emb_pad, Wk, bk2)

    gathered = _sc_gather(skill_embeds, top_idx.reshape(1, B * TOPK))

    retrieved = pl.pallas_call(
        _combine_kernel,
        out_shape=jax.ShapeDtypeStruct((B, SKILL_DIM), jnp.float32),
    )(top_vals, gathered.reshape(B, TOPK, SKILL_DIM))

    return (retrieved, top_idx)


# speedup vs baseline: 1.6548x; 1.1041x over previous
import functools
import math

import jax
import jax.numpy as jnp
from jax.experimental import pallas as pl
from jax.experimental.pallas import tpu as pltpu
from jax.experimental.pallas import tpu_sc as plsc

D_MODEL = 2048
SKILL_DIM = 128
N_SKILLS = 100000
TOPK = 8
TILE = 4096
N_PAD = ((N_SKILLS + TILE - 1) // TILE) * TILE
NUM_TILES = N_PAD // TILE
B = 1024

_NEG = float("-inf")
_BIGI = 2**30


def _dot_t(a, b):
    return jax.lax.dot_general(
        a.astype(jnp.bfloat16), b.astype(jnp.bfloat16),
        (((1,), (1,)), ((), ())), preferred_element_type=jnp.float32)


def _q_proj_kernel(h_ref, wq_ref, bq_ref, q_ref):
    q_ref[...] = _dot_t(h_ref[...], wq_ref[...]) + bq_ref[...]


def _extract8(vals, idxs=None):
    pos = jax.lax.broadcasted_iota(jnp.int32, vals.shape, 1)
    out_v, out_i = [], []
    v = vals
    for t in range(TOPK):
        m = jnp.max(v, axis=1, keepdims=True)
        p = jnp.min(jnp.where(v == m, pos, _BIGI), axis=1, keepdims=True)
        out_v.append(m)
        if idxs is None:
            out_i.append(p)
        else:
            out_i.append(jnp.min(jnp.where(pos == p, idxs, _BIGI),
                                 axis=1, keepdims=True))
        if t != TOPK - 1:
            v = jnp.where(pos == p, _NEG, v)
    return jnp.concatenate(out_v, axis=1), jnp.concatenate(out_i, axis=1)


NGRP = TILE // SKILL_DIM


def _merge8(rv, ri, t_vals, t_idx):
    cv = jnp.concatenate([rv, t_vals], axis=1)
    ci = jnp.concatenate([ri, t_idx], axis=1)
    return _extract8(cv, ci)


def _topk_kernel(q_ref, emb_ref, wk_ref, bk_ref, vals_ref, idx_ref,
                 rv_ref, ri_ref):
    j = pl.program_id(0)

    @pl.when(j == 0)
    def _():
        rv_ref[...] = jnp.full((B, TOPK), _NEG, jnp.float32)
        ri_ref[...] = jnp.zeros((B, TOPK), jnp.int32)

    keys = _dot_t(emb_ref[...], wk_ref[...]) + bk_ref[...]
    s = _dot_t(q_ref[...], keys)

    local = jax.lax.broadcasted_iota(jnp.int32, s.shape, 1)
    limit = N_SKILLS - j * TILE
    s = jnp.where(local < limit, s, _NEG)

    lane = jax.lax.broadcasted_iota(jnp.int32, (B, SKILL_DIM), 1)
    m1 = s[:, 0:SKILL_DIM]
    g1 = jnp.zeros((B, SKILL_DIM), jnp.int32)
    m2 = jnp.full((B, SKILL_DIM), _NEG, jnp.float32)
    g2 = jnp.zeros((B, SKILL_DIM), jnp.int32)
    for g in range(1, NGRP):
        x = s[:, g * SKILL_DIM:(g + 1) * SKILL_DIM]
        c1 = x > m1
        d = jnp.where(c1, m1, x)
        gd = jnp.where(c1, g1, g)
        m1 = jnp.where(c1, x, m1)
        g1 = jnp.where(c1, g, g1)
        c2 = d > m2
        m2 = jnp.where(c2, d, m2)
        g2 = jnp.where(c2, gd, g2)

    col1 = g1 * SKILL_DIM + lane
    col2 = g2 * SKILL_DIM + lane
    cvals = m1
    ccols = col1
    used = jnp.zeros((B, SKILL_DIM), jnp.int32)
    tv, ti = [], []
    for _ in range(TOPK):
        m = jnp.max(cvals, axis=1, keepdims=True)
        p = jnp.min(jnp.where(cvals == m, ccols, _BIGI), axis=1, keepdims=True)
        tv.append(m)
        ti.append(p)
        sel = (ccols == p) & (cvals == m)
        fresh = used == 0
        cvals = jnp.where(sel, jnp.where(fresh, m2, _NEG), cvals)
        ccols = jnp.where(sel, jnp.where(fresh, col2, ccols), ccols)
        used = used + sel.astype(jnp.int32)
    t_vals = jnp.concatenate(tv, axis=1)
    t_idx = jnp.concatenate(ti, axis=1) + j * TILE

    rv_old = rv_ref[...]
    ri_old = ri_ref[...]
    mv, mi = _merge8(rv_old, ri_old, t_vals, t_idx)

    tau = mv[:, TOPK - 1:TOPK]
    cnt = jnp.zeros((B, SKILL_DIM), jnp.int32)
    for g in range(NGRP):
        x = s[:, g * SKILL_DIM:(g + 1) * SKILL_DIM]
        cnt = cnt + (x >= tau).astype(jnp.int32)
    bad = jnp.max(cnt) >= 3

    @pl.when(jnp.logical_not(bad))
    def _():
        rv_ref[...] = mv
        ri_ref[...] = mi

    @pl.when(bad)
    def _():
        f_vals, f_pos = _extract8(s)
        fv, fi = _merge8(rv_old, ri_old, f_vals, f_pos + j * TILE)
        rv_ref[...] = fv
        ri_ref[...] = fi

    @pl.when(j == NUM_TILES - 1)
    def _():
        vals_ref[...] = rv_ref[...] * (1.0 / math.sqrt(SKILL_DIM))
        idx_ref[...] = ri_ref[...]


def _combine_kernel(tv_ref, g_ref, out_ref):
    tv = tv_ref[...]
    m = jnp.max(tv, axis=1, keepdims=True)
    e = jnp.exp(tv - m)
    w = e / jnp.sum(e, axis=1, keepdims=True)
    acc = jnp.zeros((B, SKILL_DIM), jnp.float32)
    for k in range(TOPK):
        acc = acc + w[:, k:k + 1] * g_ref[:, k, :]
    out_ref[...] = acc


def _sc_gather(table, indices):
    n_idx = indices.shape[1]
    window = 128
    mesh = plsc.VectorSubcoreMesh(core_axis_name="core",
                                  subcore_axis_name="subcore")

    @pl.kernel(out_type=jax.ShapeDtypeStruct((n_idx, SKILL_DIM), table.dtype),
               mesh=mesh)
    def gather_kernel(x_hbm, i_hbm, o_hbm):
        def body(i_vmem, o_vmem):
            pltpu.sync_copy(x_hbm.at[i_vmem.at[0]], o_vmem)

        pltpu.emit_pipeline(
            body,
            grid=(n_idx // window,),
            in_specs=[pl.BlockSpec((1, window), index_map=lambda i: (0, i))],
            out_specs=[pl.BlockSpec((window, SKILL_DIM),
                                    index_map=lambda i: (i, 0))],
            core_axis_name=("core", "subcore"),
            dimension_semantics=(pltpu.PARALLEL,),
        )(i_hbm, o_hbm)

    return gather_kernel(table, indices)


@jax.jit
def kernel(h, skill_embeds, Wq, bq, Wk, bk):
    h_last = h[:, -1, :]
    bq2 = bq.reshape(1, SKILL_DIM)
    bk2 = bk.reshape(1, SKILL_DIM)

    q = pl.pallas_call(
        _q_proj_kernel,
        out_shape=jax.ShapeDtypeStruct((B, SKILL_DIM), jnp.float32),
    )(h_last, Wq, bq2)

    emb_pad = jnp.pad(skill_embeds, ((0, N_PAD - N_SKILLS), (0, 0)))

    top_vals, top_idx = pl.pallas_call(
        _topk_kernel,
        grid=(NUM_TILES,),
        in_specs=[
            pl.BlockSpec((B, SKILL_DIM), lambda j: (0, 0)),
            pl.BlockSpec((TILE, SKILL_DIM), lambda j: (j, 0)),
            pl.BlockSpec((SKILL_DIM, SKILL_DIM), lambda j: (0, 0)),
            pl.BlockSpec((1, SKILL_DIM), lambda j: (0, 0)),
        ],
        out_specs=[
            pl.BlockSpec((B, TOPK), lambda j: (0, 0)),
            pl.BlockSpec((B, TOPK), lambda j: (0, 0)),
        ],
        out_shape=[
            jax.ShapeDtypeStruct((B, TOPK), jnp.float32),
            jax.ShapeDtypeStruct((B, TOPK), jnp.int32),
        ],
        scratch_shapes=[
            pltpu.VMEM((B, TOPK), jnp.float32),
            pltpu.VMEM((B, TOPK), jnp.int32),
        ],
    )(q, emb_pad, Wk, bk2)

    gathered = _sc_gather(skill_embeds, top_idx.reshape(1, B * TOPK))

    retrieved = pl.pallas_call(
        _combine_kernel,
        out_shape=jax.ShapeDtypeStruct((B, SKILL_DIM), jnp.float32),
    )(top_vals, gathered.reshape(B, TOPK, SKILL_DIM))

    return (retrieved, top_idx)


# speedup vs baseline: 1.6671x; 1.0074x over previous
import functools
import math

import jax
import jax.numpy as jnp
from jax.experimental import pallas as pl
from jax.experimental.pallas import tpu as pltpu
from jax.experimental.pallas import tpu_sc as plsc

D_MODEL = 2048
SKILL_DIM = 128
N_SKILLS = 100000
TOPK = 8
TILE = 4096
N_PAD = ((N_SKILLS + TILE - 1) // TILE) * TILE
NUM_TILES = N_PAD // TILE
B = 1024

_NEG = float("-inf")
_BIGI = 2**30


def _dot_t(a, b):
    return jax.lax.dot_general(
        a.astype(jnp.bfloat16), b.astype(jnp.bfloat16),
        (((1,), (1,)), ((), ())), preferred_element_type=jnp.float32)


def _q_proj_kernel(h_ref, wq_ref, bq_ref, q_ref):
    q_ref[...] = (_dot_t(h_ref[...], wq_ref[...])
                  + bq_ref[...]).astype(jnp.bfloat16)


def _extract8(vals, idxs=None):
    pos = jax.lax.broadcasted_iota(jnp.int32, vals.shape, 1)
    out_v, out_i = [], []
    v = vals
    for t in range(TOPK):
        m = jnp.max(v, axis=1, keepdims=True)
        p = jnp.min(jnp.where(v == m, pos, _BIGI), axis=1, keepdims=True)
        out_v.append(m)
        if idxs is None:
            out_i.append(p)
        else:
            out_i.append(jnp.min(jnp.where(pos == p, idxs, _BIGI),
                                 axis=1, keepdims=True))
        if t != TOPK - 1:
            v = jnp.where(pos == p, _NEG, v)
    return jnp.concatenate(out_v, axis=1), jnp.concatenate(out_i, axis=1)


NGRP = TILE // SKILL_DIM


def _merge8(rv, ri, t_vals, t_idx):
    cv = jnp.concatenate([rv, t_vals], axis=1)
    ci = jnp.concatenate([ri, t_idx], axis=1)
    return _extract8(cv, ci)


def _topk_kernel(q_ref, emb_ref, wk_ref, bk_ref, vals_ref, idx_ref,
                 rv_ref, ri_ref):
    j = pl.program_id(0)

    @pl.when(j == 0)
    def _():
        rv_ref[...] = jnp.full((B, TOPK), _NEG, jnp.float32)
        ri_ref[...] = jnp.zeros((B, TOPK), jnp.int32)

    keys = jax.lax.dot_general(
        emb_ref[...], wk_ref[...], (((1,), (1,)), ((), ())),
        preferred_element_type=jnp.float32) + bk_ref[...]
    s = jax.lax.dot_general(
        q_ref[...], keys.astype(jnp.bfloat16), (((1,), (1,)), ((), ())),
        preferred_element_type=jnp.float32)

    local = jax.lax.broadcasted_iota(jnp.int32, s.shape, 1)
    limit = N_SKILLS - j * TILE
    s = jnp.where(local < limit, s, _NEG)

    lane = jax.lax.broadcasted_iota(jnp.int32, (B, SKILL_DIM), 1)
    m1 = s[:, 0:SKILL_DIM]
    g1 = jnp.zeros((B, SKILL_DIM), jnp.int32)
    m2 = jnp.full((B, SKILL_DIM), _NEG, jnp.float32)
    g2 = jnp.zeros((B, SKILL_DIM), jnp.int32)
    for g in range(1, NGRP):
        x = s[:, g * SKILL_DIM:(g + 1) * SKILL_DIM]
        c1 = x > m1
        d = jnp.where(c1, m1, x)
        gd = jnp.where(c1, g1, g)
        m1 = jnp.where(c1, x, m1)
        g1 = jnp.where(c1, g, g1)
        c2 = d > m2
        m2 = jnp.where(c2, d, m2)
        g2 = jnp.where(c2, gd, g2)

    col1 = g1 * SKILL_DIM + lane
    col2 = g2 * SKILL_DIM + lane
    cvals = m1
    ccols = col1
    used = jnp.zeros((B, SKILL_DIM), jnp.int32)
    tv, ti = [], []
    for _ in range(TOPK):
        m = jnp.max(cvals, axis=1, keepdims=True)
        p = jnp.min(jnp.where(cvals == m, ccols, _BIGI), axis=1, keepdims=True)
        tv.append(m)
        ti.append(p)
        sel = (ccols == p) & (cvals == m)
        fresh = used == 0
        cvals = jnp.where(sel, jnp.where(fresh, m2, _NEG), cvals)
        ccols = jnp.where(sel, jnp.where(fresh, col2, ccols), ccols)
        used = used + sel.astype(jnp.int32)
    t_vals = jnp.concatenate(tv, axis=1)
    t_idx = jnp.concatenate(ti, axis=1) + j * TILE

    rv_old = rv_ref[...]
    ri_old = ri_ref[...]
    mv, mi = _merge8(rv_old, ri_old, t_vals, t_idx)

    tau = mv[:, TOPK - 1:TOPK]
    cnt = jnp.zeros((B, SKILL_DIM), jnp.int32)
    for g in range(NGRP):
        x = s[:, g * SKILL_DIM:(g + 1) * SKILL_DIM]
        cnt = cnt + (x >= tau).astype(jnp.int32)
    bad = jnp.max(cnt) >= 3

    @pl.when(jnp.logical_not(bad))
    def _():
        rv_ref[...] = mv
        ri_ref[...] = mi

    @pl.when(bad)
    def _():
        f_vals, f_pos = _extract8(s)
        fv, fi = _merge8(rv_old, ri_old, f_vals, f_pos + j * TILE)
        rv_ref[...] = fv
        ri_ref[...] = fi

    @pl.when(j == NUM_TILES - 1)
    def _():
        vals_ref[...] = rv_ref[...] * (1.0 / math.sqrt(SKILL_DIM))
        idx_ref[...] = ri_ref[...]


def _combine_kernel(tv_ref, g_ref, out_ref):
    tv = tv_ref[...]
    m = jnp.max(tv, axis=1, keepdims=True)
    e = jnp.exp(tv - m)
    w = e / jnp.sum(e, axis=1, keepdims=True)
    acc = jnp.zeros((B, SKILL_DIM), jnp.float32)
    for k in range(TOPK):
        acc = acc + w[:, k:k + 1] * g_ref[:, k, :]
    out_ref[...] = acc


def _sc_gather(table, indices):
    n_idx = indices.shape[1]
    window = 128
    mesh = plsc.VectorSubcoreMesh(core_axis_name="core",
                                  subcore_axis_name="subcore")

    @pl.kernel(out_type=jax.ShapeDtypeStruct((n_idx, SKILL_DIM), table.dtype),
               mesh=mesh)
    def gather_kernel(x_hbm, i_hbm, o_hbm):
        def body(i_vmem, o_vmem):
            pltpu.sync_copy(x_hbm.at[i_vmem.at[0]], o_vmem)

        pltpu.emit_pipeline(
            body,
            grid=(n_idx // window,),
            in_specs=[pl.BlockSpec((1, window), index_map=lambda i: (0, i))],
            out_specs=[pl.BlockSpec((window, SKILL_DIM),
                                    index_map=lambda i: (i, 0))],
            core_axis_name=("core", "subcore"),
            dimension_semantics=(pltpu.PARALLEL,),
        )(i_hbm, o_hbm)

    return gather_kernel(table, indices)


@jax.jit
def kernel(h, skill_embeds, Wq, bq, Wk, bk):
    h_last = h[:, -1, :]
    bq2 = bq.reshape(1, SKILL_DIM)
    bk2 = bk.reshape(1, SKILL_DIM)

    q = pl.pallas_call(
        _q_proj_kernel,
        out_shape=jax.ShapeDtypeStruct((B, SKILL_DIM), jnp.bfloat16),
    )(h_last, Wq, bq2)

    emb_bf = skill_embeds.astype(jnp.bfloat16)
    wk_bf = Wk.astype(jnp.bfloat16)

    top_vals, top_idx = pl.pallas_call(
        _topk_kernel,
        grid=(NUM_TILES,),
        in_specs=[
            pl.BlockSpec((B, SKILL_DIM), lambda j: (0, 0)),
            pl.BlockSpec((TILE, SKILL_DIM), lambda j: (j, 0)),
            pl.BlockSpec((SKILL_DIM, SKILL_DIM), lambda j: (0, 0)),
            pl.BlockSpec((1, SKILL_DIM), lambda j: (0, 0)),
        ],
        out_specs=[
            pl.BlockSpec((B, TOPK), lambda j: (0, 0)),
            pl.BlockSpec((B, TOPK), lambda j: (0, 0)),
        ],
        out_shape=[
            jax.ShapeDtypeStruct((B, TOPK), jnp.float32),
            jax.ShapeDtypeStruct((B, TOPK), jnp.int32),
        ],
        scratch_shapes=[
            pltpu.VMEM((B, TOPK), jnp.float32),
            pltpu.VMEM((B, TOPK), jnp.int32),
        ],
    )(q, emb_bf, wk_bf, bk2)

    gathered = _sc_gather(skill_embeds, top_idx.reshape(1, B * TOPK))

    retrieved = pl.pallas_call(
        _combine_kernel,
        out_shape=jax.ShapeDtypeStruct((B, SKILL_DIM), jnp.float32),
    )(top_vals, gathered.reshape(B, TOPK, SKILL_DIM))

    return (retrieved, top_idx)
